# baseline (speedup 1.0000x reference)
import functools

import jax
import jax.numpy as jnp
from jax import lax
from jax.experimental import pallas as pl
from jax.experimental.pallas import tpu as pltpu
from jax.experimental.pallas import tpu_sc as plsc

N = 10000
NP = 10112
E = 320000
NC = 2
NS = 16
CHE = 8192
NCH = 20
EPH = NCH * CHE
EP = NC * EPH
L = 16


@functools.cache
def _sc_edge_pass(width):
    cpt = width // NS
    mesh = plsc.VectorSubcoreMesh(
        core_axis_name="c", subcore_axis_name="s", num_cores=NC, num_subcores=NS
    )

    @functools.partial(
        pl.kernel,
        out_type=jax.ShapeDtypeStruct((NC, width, NP), jnp.float32),
        mesh=mesh,
        scratch_types=[
            pltpu.VMEM((cpt, NP), jnp.float32),
            pltpu.VMEM((cpt, NP), jnp.float32),
            pltpu.VMEM((CHE,), jnp.int32),
            pltpu.VMEM((CHE,), jnp.int32),
            pltpu.VMEM((CHE,), jnp.int32),
            pltpu.VMEM((CHE,), jnp.int32),
            [pltpu.SemaphoreType.DMA] * 2,
            [pltpu.SemaphoreType.DMA] * 2,
        ],
        compiler_params=pltpu.CompilerParams(
            use_tc_tiling_on_sc=False, needs_layout_passes=False
        ),
    )
    def sc(tbl_hbm, src_hbm, dst_hbm, out_hbm,
           tbl_v, acc_v, se0, se1, de0, de1, ssems, dsems):
        c = lax.axis_index("c")
        s = lax.axis_index("s")
        ses = [se0, se1]
        des = [de0, de1]
        ebase = c * EPH
        rb = s * cpt
        pltpu.sync_copy(tbl_hbm.at[pl.ds(rb, cpt)], tbl_v)
        zeros = jnp.zeros((L,), jnp.float32)

        def zb(i, carry):
            for k in range(cpt):
                acc_v[k, pl.ds(i * L, L)] = zeros
            return carry

        lax.fori_loop(0, NP // L, zb, 0)

        def fire(ch, b):
            off = ebase + ch * CHE
            pltpu.async_copy(src_hbm.at[pl.ds(off, CHE)], ses[b], ssems[b])
            pltpu.async_copy(dst_hbm.at[pl.ds(off, CHE)], des[b], dsems[b])

        def wait(ch, b):
            off = ebase + ch * CHE
            pltpu.make_async_copy(src_hbm.at[pl.ds(off, CHE)],
                                  ses[b], ssems[b]).wait()
            pltpu.make_async_copy(dst_hbm.at[pl.ds(off, CHE)],
                                  des[b], dsems[b]).wait()

        def process(ch, b):
            wait(ch, b)
            se = ses[b]
            de = des[b]

            UNR = 8

            def inner(i, carry):
                svs = [se[pl.ds((UNR * i + u) * L, L)] for u in range(UNR)]
                dvs = [de[pl.ds((UNR * i + u) * L, L)] for u in range(UNR)]
                gs = []
                for u in range(UNR):
                    for k in range(cpt):
                        kv = jnp.full((L,), k, jnp.int32)
                        gs.append(plsc.load_gather(tbl_v, [kv, svs[u]]))
                for u in range(UNR):
                    for k in range(cpt):
                        kv = jnp.full((L,), k, jnp.int32)
                        plsc.addupdate_scatter(acc_v, [kv, dvs[u]],
                                               gs[u * cpt + k])
                return carry

            lax.fori_loop(0, CHE // (UNR * L), inner, 0)

        fire(0, 0)

        def grp(g, carry):
            j0 = 2 * g
            fire(jnp.minimum(j0 + 1, NCH - 1), 1)
            process(j0, 0)
            fire(jnp.minimum(j0 + 2, NCH - 1), 0)
            process(j0 + 1, 1)
            return carry

        lax.fori_loop(0, NCH // 2, grp, 0)
        wait(NCH - 1, 0)
        pltpu.sync_copy(acc_v, out_hbm.at[c, pl.ds(rb, cpt)])

    return sc


@functools.cache
def _sc_degree_count():
    epw = EP // (NC * NS)
    mesh = plsc.VectorSubcoreMesh(
        core_axis_name="c", subcore_axis_name="s", num_cores=NC, num_subcores=NS
    )

    @functools.partial(
        pl.kernel,
        out_type=jax.ShapeDtypeStruct((NC * NS, NP), jnp.float32),
        mesh=mesh,
        scratch_types=[
            pltpu.VMEM((NP,), jnp.float32),
            pltpu.VMEM((epw,), jnp.int32),
        ],
        compiler_params=pltpu.CompilerParams(
            use_tc_tiling_on_sc=False, needs_layout_passes=False
        ),
    )
    def sc(dst_hbm, out_hbm, cnt_v, de_v):
        c = lax.axis_index("c")
        s = lax.axis_index("s")
        w = c * NS + s
        pltpu.sync_copy(dst_hbm.at[pl.ds(w * epw, epw)], de_v)
        zeros = jnp.zeros((L,), jnp.float32)

        def zb(i, carry):
            cnt_v[pl.ds(i * L, L)] = zeros
            return carry

        lax.fori_loop(0, NP // L, zb, 0)
        ones = jnp.ones((L,), jnp.float32)

        def inner(i, carry):
            for u in range(4):
                dv = de_v[pl.ds((4 * i + u) * L, L)]
                plsc.addupdate_scatter(cnt_v, [dv], ones)
            return carry

        lax.fori_loop(0, epw // (4 * L), inner, 0)
        pltpu.sync_copy(cnt_v, out_hbm.at[w])

    return sc


def _tc1_body(xt_ref, wl_ref, wr_ref, b1_ref, y1t_ref, xr1t_ref):
    xt = xt_ref[...]
    y1t_ref[...] = jnp.dot(wl_ref[...], xt, preferred_element_type=jnp.float32)
    xr1t_ref[...] = jnp.dot(wr_ref[...], xt,
                            preferred_element_type=jnp.float32) + b1_ref[...]


def _tc2_body(acc_ref, cnt_ref, xr1t_ref, wl2_ref, wr2_ref, b2_ref,
              y2t_ref, hr2t_ref, inv8_ref):
    a = acc_ref[0] + acc_ref[1]
    cnt = jnp.sum(cnt_ref[...], axis=0, keepdims=True)
    inv = 1.0 / jnp.maximum(cnt, 1.0)
    h = jnp.maximum(a * inv + xr1t_ref[...], 0.0)
    y2t_ref[...] = jnp.dot(wl2_ref[...], h, preferred_element_type=jnp.float32)
    hr2t_ref[...] = jnp.dot(wr2_ref[...], h,
                            preferred_element_type=jnp.float32) + b2_ref[...]
    inv8_ref[...] = jnp.broadcast_to(inv, (8, NP))


def _tc3_body(acc2_ref, hr2t_ref, inv8_ref, w3_ref, b3_ref, out_ref):
    a2 = acc2_ref[0] + acc2_ref[1]
    h2 = a2 * inv8_ref[...][0:1, :] + hr2t_ref[...]
    out_ref[...] = jnp.dot(w3_ref[...], h2,
                           preferred_element_type=jnp.float32) + b3_ref[...]


def kernel(x, edge_index, Wl1, Wr1, b1, Wl2, Wr2, b2, W3, b3):
    xt = jnp.pad(x, ((0, NP - N), (0, 0))).T
    ei = edge_index.astype(jnp.int32)
    pad = jnp.full((EP - E,), N, jnp.int32)
    src = jnp.concatenate([ei[0], pad])
    dst = jnp.concatenate([ei[1], pad])

    cnt = _sc_degree_count()(dst)

    y1t, xr1t = pl.pallas_call(
        _tc1_body,
        out_shape=(jax.ShapeDtypeStruct((64, NP), jnp.float32),
                   jax.ShapeDtypeStruct((64, NP), jnp.float32)),
    )(xt, Wl1.T, Wr1.T, b1.reshape(64, 1))

    acc1 = _sc_edge_pass(64)(y1t, src, dst)

    y2t, hr2t, inv8 = pl.pallas_call(
        _tc2_body,
        out_shape=(jax.ShapeDtypeStruct((32, NP), jnp.float32),
                   jax.ShapeDtypeStruct((32, NP), jnp.float32),
                   jax.ShapeDtypeStruct((8, NP), jnp.float32)),
    )(acc1, cnt, xr1t, Wl2.T, Wr2.T, b2.reshape(32, 1))

    acc2 = _sc_edge_pass(32)(y2t, src, dst)

    w3t = jnp.pad(W3.T, ((0, 6), (0, 0)))
    b3t = jnp.pad(b3, (0, 6)).reshape(8, 1)
    outt = pl.pallas_call(
        _tc3_body,
        out_shape=jax.ShapeDtypeStruct((8, NP), jnp.float32),
    )(acc2, hr2t, inv8, w3t, b3t)
    return outt[:2, :N].T


# speedup vs baseline: 1.0213x; 1.0213x over previous
import functools

import jax
import jax.numpy as jnp
from jax import lax
from jax.experimental import pallas as pl
from jax.experimental.pallas import tpu as pltpu
from jax.experimental.pallas import tpu_sc as plsc

N = 10000
NP = 10112
E = 320000
NC = 2
NS = 16
CHE = 8192
NCH = 20
EPH = NCH * CHE
EP = NC * EPH
L = 16


@functools.cache
def _sc_edge_pass(width):
    cpt = width // NS
    mesh = plsc.VectorSubcoreMesh(
        core_axis_name="c", subcore_axis_name="s", num_cores=NC, num_subcores=NS
    )

    @functools.partial(
        pl.kernel,
        out_type=jax.ShapeDtypeStruct((NC, width, NP), jnp.float32),
        mesh=mesh,
        scratch_types=[
            pltpu.VMEM((cpt, NP), jnp.float32),
            pltpu.VMEM((cpt, NP), jnp.float32),
            pltpu.VMEM((CHE,), jnp.int32),
            pltpu.VMEM((CHE,), jnp.int32),
            pltpu.VMEM((CHE,), jnp.int32),
            pltpu.VMEM((CHE,), jnp.int32),
            [pltpu.SemaphoreType.DMA] * 2,
            [pltpu.SemaphoreType.DMA] * 2,
        ],
        compiler_params=pltpu.CompilerParams(
            use_tc_tiling_on_sc=False, needs_layout_passes=False
        ),
    )
    def sc(tbl_hbm, src_hbm, dst_hbm, out_hbm,
           tbl_v, acc_v, se0, se1, de0, de1, ssems, dsems):
        c = lax.axis_index("c")
        s = lax.axis_index("s")
        ses = [se0, se1]
        des = [de0, de1]
        ebase = c * EPH
        rb = s * cpt

        def fire(ch, b):
            off = ebase + ch * CHE
            pltpu.async_copy(src_hbm.at[pl.ds(off, CHE)], ses[b], ssems[b])
            pltpu.async_copy(dst_hbm.at[pl.ds(off, CHE)], des[b], dsems[b])

        fire(0, 0)
        pltpu.sync_copy(tbl_hbm.at[pl.ds(rb, cpt)], tbl_v)
        zeros = jnp.zeros((L,), jnp.float32)

        def zb(i, carry):
            for k in range(cpt):
                acc_v[k, pl.ds(i * L, L)] = zeros
            return carry

        lax.fori_loop(0, NP // L, zb, 0)

        def wait(ch, b):
            off = ebase + ch * CHE
            pltpu.make_async_copy(src_hbm.at[pl.ds(off, CHE)],
                                  ses[b], ssems[b]).wait()
            pltpu.make_async_copy(dst_hbm.at[pl.ds(off, CHE)],
                                  des[b], dsems[b]).wait()

        def process(ch, b):
            wait(ch, b)
            se = ses[b]
            de = des[b]

            UNR = 4

            def inner(i, carry):
                svs = [se[pl.ds((UNR * i + u) * L, L)] for u in range(UNR)]
                dvs = [de[pl.ds((UNR * i + u) * L, L)] for u in range(UNR)]
                gs = []
                for u in range(UNR):
                    for k in range(cpt):
                        kv = jnp.full((L,), k, jnp.int32)
                        gs.append(plsc.load_gather(tbl_v, [kv, svs[u]]))
                for u in range(UNR):
                    for k in range(cpt):
                        kv = jnp.full((L,), k, jnp.int32)
                        plsc.addupdate_scatter(acc_v, [kv, dvs[u]],
                                               gs[u * cpt + k])
                return carry

            lax.fori_loop(0, CHE // (UNR * L), inner, 0)

        def grp(g, carry):
            j0 = 2 * g
            fire(jnp.minimum(j0 + 1, NCH - 1), 1)
            process(j0, 0)
            fire(jnp.minimum(j0 + 2, NCH - 1), 0)
            process(j0 + 1, 1)
            return carry

        lax.fori_loop(0, NCH // 2, grp, 0)
        wait(NCH - 1, 0)
        pltpu.sync_copy(acc_v, out_hbm.at[c, pl.ds(rb, cpt)])

    return sc


@functools.cache
def _sc_degree_count():
    epw = EP // (NC * NS)
    mesh = plsc.VectorSubcoreMesh(
        core_axis_name="c", subcore_axis_name="s", num_cores=NC, num_subcores=NS
    )

    @functools.partial(
        pl.kernel,
        out_type=jax.ShapeDtypeStruct((NC * NS, NP), jnp.float32),
        mesh=mesh,
        scratch_types=[
            pltpu.VMEM((NP,), jnp.float32),
            pltpu.VMEM((epw,), jnp.int32),
        ],
        compiler_params=pltpu.CompilerParams(
            use_tc_tiling_on_sc=False, needs_layout_passes=False
        ),
    )
    def sc(dst_hbm, out_hbm, cnt_v, de_v):
        c = lax.axis_index("c")
        s = lax.axis_index("s")
        w = c * NS + s
        pltpu.sync_copy(dst_hbm.at[pl.ds(w * epw, epw)], de_v)
        zeros = jnp.zeros((L,), jnp.float32)

        def zb(i, carry):
            cnt_v[pl.ds(i * L, L)] = zeros
            return carry

        lax.fori_loop(0, NP // L, zb, 0)
        ones = jnp.ones((L,), jnp.float32)

        def inner(i, carry):
            for u in range(4):
                dv = de_v[pl.ds((4 * i + u) * L, L)]
                plsc.addupdate_scatter(cnt_v, [dv], ones)
            return carry

        lax.fori_loop(0, epw // (4 * L), inner, 0)
        pltpu.sync_copy(cnt_v, out_hbm.at[w])

    return sc


def _tc1_body(xt_ref, wl_ref, wr_ref, b1_ref, y1t_ref, xr1t_ref):
    xt = xt_ref[...]
    y1t_ref[...] = jnp.dot(wl_ref[...], xt, preferred_element_type=jnp.float32)
    xr1t_ref[...] = jnp.dot(wr_ref[...], xt,
                            preferred_element_type=jnp.float32) + b1_ref[...]


def _tc2_body(acc_ref, cnt_ref, xr1t_ref, wl2_ref, wr2_ref, b2_ref,
              y2t_ref, hr2t_ref, inv8_ref):
    a = acc_ref[0] + acc_ref[1]
    cnt = jnp.sum(cnt_ref[...], axis=0, keepdims=True)
    inv = 1.0 / jnp.maximum(cnt, 1.0)
    h = jnp.maximum(a * inv + xr1t_ref[...], 0.0)
    y2t_ref[...] = jnp.dot(wl2_ref[...], h, preferred_element_type=jnp.float32)
    hr2t_ref[...] = jnp.dot(wr2_ref[...], h,
                            preferred_element_type=jnp.float32) + b2_ref[...]
    inv8_ref[...] = jnp.broadcast_to(inv, (8, NP))


def _tc3_body(acc2_ref, hr2t_ref, inv8_ref, w3_ref, b3_ref, out_ref):
    a2 = acc2_ref[0] + acc2_ref[1]
    h2 = a2 * inv8_ref[...][0:1, :] + hr2t_ref[...]
    out_ref[...] = jnp.dot(w3_ref[...], h2,
                           preferred_element_type=jnp.float32) + b3_ref[...]


def kernel(x, edge_index, Wl1, Wr1, b1, Wl2, Wr2, b2, W3, b3):
    xt = jnp.pad(x, ((0, NP - N), (0, 0))).T
    ei = edge_index.astype(jnp.int32)
    pad = jnp.full((EP - E,), N, jnp.int32)
    src = jnp.concatenate([ei[0], pad])
    dst = jnp.concatenate([ei[1], pad])

    cnt = _sc_degree_count()(dst)

    y1t, xr1t = pl.pallas_call(
        _tc1_body,
        out_shape=(jax.ShapeDtypeStruct((64, NP), jnp.float32),
                   jax.ShapeDtypeStruct((64, NP), jnp.float32)),
    )(xt, Wl1.T, Wr1.T, b1.reshape(64, 1))

    acc1 = _sc_edge_pass(64)(y1t, src, dst)

    y2t, hr2t, inv8 = pl.pallas_call(
        _tc2_body,
        out_shape=(jax.ShapeDtypeStruct((32, NP), jnp.float32),
                   jax.ShapeDtypeStruct((32, NP), jnp.float32),
                   jax.ShapeDtypeStruct((8, NP), jnp.float32)),
    )(acc1, cnt, xr1t, Wl2.T, Wr2.T, b2.reshape(32, 1))

    acc2 = _sc_edge_pass(32)(y2t, src, dst)

    w3t = jnp.pad(W3.T, ((0, 6), (0, 0)))
    b3t = jnp.pad(b3, (0, 6)).reshape(8, 1)
    outt = pl.pallas_call(
        _tc3_body,
        out_shape=jax.ShapeDtypeStruct((8, NP), jnp.float32),
    )(acc2, hr2t, inv8, w3t, b3t)
    return outt[:2, :N].T
